# Initial kernel scaffold; baseline (speedup 1.0000x reference)
import jax
import jax.numpy as jnp
from jax.experimental import pallas as pl


def kernel(x, edge_index, W1, a1_src, a1_dst, b1, W2, a2_src, a2_dst, b2):
    raise NotImplementedError("write your pallas kernel here")


# speedup vs baseline: 2.7124x; 2.7124x over previous
import jax
import jax.numpy as jnp
from jax import lax
from jax.experimental import pallas as pl
from jax.experimental.pallas import tpu as pltpu
from jax.experimental.pallas import tpu_sc as plsc

N_NODES = 10000
NP = 10240
E_RAW = 160000
EL = E_RAW + N_NODES
B = 64
NTILES = 32
EP = 180224
ET = EP // NTILES
NB = ET // B
STRIPE = NP // 16


def _mm_body(x_ref, w_ref, o_ref):
    o_ref[0] = jnp.dot(x_ref[...], w_ref[...],
                       preferred_element_type=jnp.float32)


def _matmul_chunks(x, w, n_chunks):
    m, k = x.shape
    grid = (m // 256, n_chunks)
    return pl.pallas_call(
        _mm_body,
        grid=grid,
        in_specs=[
            pl.BlockSpec((256, k), lambda i, c: (i, 0)),
            pl.BlockSpec((k, 128), lambda i, c: (0, c)),
        ],
        out_specs=pl.BlockSpec((1, 256, 128), lambda i, c: (c, i, 0)),
        out_shape=jax.ShapeDtypeStruct((n_chunks, m, 128), jnp.float32),
    )(x, w)


def _mm2_body(agg_ref, den_ref, b_ref, w_ref, o_ref):
    c = pl.program_id(1)
    a = agg_ref[0, 0] + agg_ref[1, 0]
    den = den_ref[0, 0][:, 0:1] + den_ref[1, 0][:, 0:1] + 1e-16
    h = a / den + b_ref[0, 0][None, :]
    h = jnp.maximum(h, 0.0)
    contrib = jnp.dot(h, w_ref[0], preferred_element_type=jnp.float32)

    @pl.when(c == 0)
    def _():
        col = lax.broadcasted_iota(jnp.int32, (256, 128), 1)
        o_ref[...] = contrib + jnp.where(col == 66, 1.0, 0.0)

    @pl.when(c > 0)
    def _():
        o_ref[...] += contrib


def _matmul2(agg1, b1r, w2r):
    grid = (NP // 256, 4)
    return pl.pallas_call(
        _mm2_body,
        grid=grid,
        in_specs=[
            pl.BlockSpec((2, 1, 256, 128), lambda i, c: (0, c, i, 0)),
            pl.BlockSpec((2, 1, 256, 128), lambda i, c: (0, 4, i, 0)),
            pl.BlockSpec((1, 1, 128), lambda i, c: (c, 0, 0)),
            pl.BlockSpec((1, 128, 128), lambda i, c: (c, 0, 0)),
        ],
        out_specs=pl.BlockSpec((256, 128), lambda i, c: (i, 0)),
        out_shape=jax.ShapeDtypeStruct((NP, 128), jnp.float32),
    )(agg1, agg1, b1r, w2r)


def _fin_body(agg_ref, b_ref, o_ref):
    a = agg_ref[0, 0] + agg_ref[1, 0]
    den = a[:, 66:67] + 1e-16
    o_ref[...] = a[:, :64] / den + b_ref[...][None, :]


def _final(agg2, b2):
    return pl.pallas_call(
        _fin_body,
        grid=(NP // 256,),
        in_specs=[
            pl.BlockSpec((2, 1, 256, 128), lambda i: (0, 0, i, 0)),
            pl.BlockSpec((64,), lambda i: (0,)),
        ],
        out_specs=pl.BlockSpec((256, 64), lambda i: (i, 0)),
        out_shape=jax.ShapeDtypeStruct((NP, 64), jnp.float32),
    )(agg2, b2)


def _make_agg(ch_count, den_chunk, n_pass):
    n_out = ch_count + (1 if den_chunk else 0)
    acc_rows = NP // n_pass
    acc_alloc = acc_rows + (16 if n_pass > 1 else 0)
    stripe = acc_rows // 16

    def body(*refs):
        h_refs = refs[:ch_count]
        s_hbm, d_hbm, src2_hbm, dst2_hbm, z_hbm, out_hbm = \
            refs[ch_count:ch_count + 6]
        (s_v, d_v, w_v, src2_v, dst2_v, gbuf, obuf, acc,
         sg0, sg1, ss0, ss1) = refs[ch_count + 6:]

        core = lax.axis_index("c")
        sub = lax.axis_index("s")
        wid = sub * 2 + core
        rbase = pl.multiple_of(wid * NB, 8)

        pltpu.sync_copy(s_hbm, s_v)
        pltpu.sync_copy(d_hbm, d_v)
        pltpu.sync_copy(src2_hbm.at[pl.ds(rbase, NB)], src2_v)
        pltpu.sync_copy(dst2_hbm.at[pl.ds(rbase, NB)], dst2_v)

        def wbody(g, _):
            for l in range(B // 16):
                si = src2_v[g, pl.ds(l * 16, 16)]
                di = dst2_v[g, pl.ds(l * 16, 16)]
                sv = plsc.load_gather(s_v, [si])
                dv = plsc.load_gather(d_v, [di])
                e = sv + dv
                e = jnp.where(e >= 0.0, e, 0.2 * e)
                w_v[pl.ds(g * B + l * 16, 16)] = jnp.exp(e)
            return 0

        lax.fori_loop(0, NB, wbody, 0)

        ii = lax.iota(jnp.int32, 16)
        e0 = jnp.where(ii == 0, 1.0, 0.0).astype(jnp.float32)
        zero16 = jnp.zeros((16,), jnp.float32)
        sems_g = (sg0, sg1)
        sems_s = (ss0, ss1)

        def zero_stripe():
            pltpu.sync_copy(z_hbm.at[pl.ds(0, stripe)],
                            acc.at[pl.ds(sub * stripe, stripe)])
            plsc.subcore_barrier()

        def drain_stripe(ch, p):
            off = pl.multiple_of(
                core * (n_out * NP) + ch * NP + p * acc_rows + sub * stripe,
                8)
            pltpu.sync_copy(acc.at[pl.ds(sub * stripe, stripe)],
                            out_hbm.at[pl.ds(off, stripe)])
            plsc.subcore_barrier()

        for p in range(n_pass):
            if n_pass > 1:
                pltpu.sync_copy(dst2_hbm.at[pl.ds(rbase, NB)], dst2_v)

                def tbody(g, _):
                    for l in range(B // 16):
                        di = dst2_v[g, pl.ds(l * 16, 16)]
                        dc = di - p * acc_rows
                        ok = (dc >= 0) & (dc < acc_rows)
                        dst2_v[g, pl.ds(l * 16, 16)] = jnp.where(
                            ok, dc, acc_rows)
                    return 0

                lax.fori_loop(0, NB, tbody, 0)

            for ch in range(ch_count):
                h_hbm = h_refs[ch]
                zero_stripe()

                for b in (0, 1):
                    pltpu.async_copy(
                        h_hbm.at[src2_v.at[b]],
                        gbuf.at[pl.ds(b * B, B)], sems_g[b])

                def batch_pair(g2, _):
                    for b in (0, 1):
                        g = 2 * g2 + b
                        pltpu.make_async_copy(
                            h_hbm.at[src2_v.at[g]],
                            gbuf.at[pl.ds(b * B, B)], sems_g[b]).wait()

                        @pl.when(g >= 2)
                        def _():
                            pltpu.make_async_copy(
                                obuf.at[pl.ds(b * B, B)],
                                acc.at[dst2_v.at[g - 2]],
                                sems_s[b]).wait()

                        def row(i, _):
                            widx = jnp.broadcast_to(
                                (g * B + i).astype(jnp.int32), (16,))
                            ws = plsc.load_gather(w_v, [widx])
                            r = b * B + i
                            for k in range(8):
                                v = gbuf[r, pl.ds(k * 16, 16)]
                                obuf[r, pl.ds(k * 16, 16)] = v * ws
                            return 0

                        lax.fori_loop(0, B, row, 0)

                        pltpu.async_copy(
                            obuf.at[pl.ds(b * B, B)],
                            acc.at[dst2_v.at[g]], sems_s[b], add=True)

                        @pl.when(g + 2 < NB)
                        def _():
                            pltpu.async_copy(
                                h_hbm.at[src2_v.at[g + 2]],
                                gbuf.at[pl.ds(b * B, B)], sems_g[b])
                    return 0

                lax.fori_loop(0, NB // 2, batch_pair, 0)

                for b in (0, 1):
                    pltpu.make_async_copy(
                        obuf.at[pl.ds(b * B, B)],
                        acc.at[dst2_v.at[NB - 2 + b]], sems_s[b]).wait()
                plsc.subcore_barrier()
                drain_stripe(ch, p)

            if den_chunk:
                zero_stripe()

                def zrow(i, _):
                    for k in range(1, 8):
                        obuf[i, pl.ds(k * 16, 16)] = zero16
                    return 0

                lax.fori_loop(0, 2 * B, zrow, 0)

                def dbatch_pair(g2, _):
                    for b in (0, 1):
                        g = 2 * g2 + b

                        @pl.when(g >= 2)
                        def _():
                            pltpu.make_async_copy(
                                obuf.at[pl.ds(b * B, B)],
                                acc.at[dst2_v.at[g - 2]],
                                sems_s[b]).wait()

                        def drow(i, _):
                            widx = jnp.broadcast_to(
                                (g * B + i).astype(jnp.int32), (16,))
                            ws = plsc.load_gather(w_v, [widx])
                            obuf[b * B + i, pl.ds(0, 16)] = ws * e0
                            return 0

                        lax.fori_loop(0, B, drow, 0)

                        pltpu.async_copy(
                            obuf.at[pl.ds(b * B, B)],
                            acc.at[dst2_v.at[g]], sems_s[b], add=True)
                    return 0

                lax.fori_loop(0, NB // 2, dbatch_pair, 0)

                for b in (0, 1):
                    pltpu.make_async_copy(
                        obuf.at[pl.ds(b * B, B)],
                        acc.at[dst2_v.at[NB - 2 + b]], sems_s[b]).wait()
                plsc.subcore_barrier()
                drain_stripe(ch_count, p)

    mesh = plsc.VectorSubcoreMesh(core_axis_name="c", subcore_axis_name="s")
    return pl.kernel(
        body,
        out_type=jax.ShapeDtypeStruct((2 * n_out * NP, 128), jnp.float32),
        mesh=mesh,
        compiler_params=pltpu.CompilerParams(needs_layout_passes=False),
        scratch_types=[
            pltpu.VMEM((NP,), jnp.float32),
            pltpu.VMEM((NP,), jnp.float32),
            pltpu.VMEM((ET,), jnp.float32),
            pltpu.VMEM((NB, B), jnp.int32),
            pltpu.VMEM((NB, B), jnp.int32),
            pltpu.VMEM((2 * B, 128), jnp.float32),
            pltpu.VMEM((2 * B, 128), jnp.float32),
            pltpu.VMEM_SHARED((acc_alloc, 128), jnp.float32),
            pltpu.SemaphoreType.DMA,
            pltpu.SemaphoreType.DMA,
            pltpu.SemaphoreType.DMA,
            pltpu.SemaphoreType.DMA,
        ],
    )


@jax.jit
def _run(x, src2, dst2, W1ext, b1r, W2ext3, b2):
    zeros = jnp.zeros((STRIPE, 128), jnp.float32)

    h1 = _matmul_chunks(x, W1ext, 5)
    s1 = h1[4, :, 0]
    d1 = h1[4, :, 1]
    agg1_fn = _make_agg(4, True, 2)
    agg1 = agg1_fn(h1[0], h1[1], h1[2], h1[3], s1, d1, src2, dst2, zeros)
    agg1 = agg1.reshape(2, 5, NP, 128)

    out2 = _matmul2(agg1, b1r, W2ext3)
    s2 = out2[:, 64]
    d2 = out2[:, 65]
    agg2_fn = _make_agg(1, False, 2)
    agg2 = agg2_fn(out2, s2, d2, src2, dst2, zeros)
    agg2 = agg2.reshape(2, 1, NP, 128)

    out = _final(agg2, b2)
    return out[:N_NODES]


def kernel(x, edge_index, W1, a1_src, a1_dst, b1, W2, a2_src, a2_dst, b2):
    ei = edge_index.astype(jnp.int32)
    loop = jnp.arange(N_NODES, dtype=jnp.int32)
    src = jnp.concatenate([ei[0], loop])
    dst = jnp.concatenate([ei[1], loop])
    src2 = jnp.pad(src, (0, EP - EL)).reshape(EP // B, B)
    dst2 = jnp.pad(dst, (0, EP - EL),
                   constant_values=N_NODES).reshape(EP // B, B)

    xp = jnp.pad(x, ((0, NP - N_NODES), (0, 0)))

    W1ext = jnp.zeros((256, 640), jnp.float32)
    W1ext = W1ext.at[:, :512].set(W1)
    W1ext = W1ext.at[:, 512].set(W1 @ a1_src)
    W1ext = W1ext.at[:, 513].set(W1 @ a1_dst)

    W2ext = jnp.zeros((512, 128), jnp.float32)
    W2ext = W2ext.at[:, :64].set(W2)
    W2ext = W2ext.at[:, 64].set(W2 @ a2_src)
    W2ext = W2ext.at[:, 65].set(W2 @ a2_dst)
    W2ext3 = W2ext.reshape(4, 128, 128)

    b1r = b1.reshape(4, 1, 128)

    return _run(xp, src2, dst2, W1ext, b1r, W2ext3, b2)


# speedup vs baseline: 3.2473x; 1.1972x over previous
import jax
import jax.numpy as jnp
from jax import lax
from jax.experimental import pallas as pl
from jax.experimental.pallas import tpu as pltpu
from jax.experimental.pallas import tpu_sc as plsc

N_NODES = 10000
NP = 10240
HALF = NP // 2
E_RAW = 160000
EL = E_RAW + N_NODES
B = 64
EP = 180224
ET = EP // 32
NB = ET // B
SEGB = 96
SEGCAP = SEGB * B
STRIPE = HALF // 16


def _mm_body(x_ref, w_ref, o_ref):
    o_ref[0] = jnp.dot(x_ref[...], w_ref[...],
                       preferred_element_type=jnp.float32)


def _matmul_chunks(x, w, n_chunks):
    m, k = x.shape
    grid = (m // 256, n_chunks)
    return pl.pallas_call(
        _mm_body,
        grid=grid,
        in_specs=[
            pl.BlockSpec((256, k), lambda i, c: (i, 0)),
            pl.BlockSpec((k, 128), lambda i, c: (0, c)),
        ],
        out_specs=pl.BlockSpec((1, 256, 128), lambda i, c: (c, i, 0)),
        out_shape=jax.ShapeDtypeStruct((n_chunks, m, 128), jnp.float32),
    )(x, w)


def _mm2_body(agg_ref, den_ref, b_ref, w_ref, o_ref):
    c = pl.program_id(1)
    a = agg_ref[0]
    den = den_ref[0][:, 0:1] + 1e-16
    h = a / den + b_ref[0, 0][None, :]
    h = jnp.maximum(h, 0.0)
    contrib = jnp.dot(h, w_ref[0], preferred_element_type=jnp.float32)

    @pl.when(c == 0)
    def _():
        col = lax.broadcasted_iota(jnp.int32, (256, 128), 1)
        o_ref[...] = contrib + jnp.where(col == 66, 1.0, 0.0)

    @pl.when(c > 0)
    def _():
        o_ref[...] += contrib


def _matmul2(agg1, b1r, w2r):
    grid = (NP // 256, 4)
    return pl.pallas_call(
        _mm2_body,
        grid=grid,
        in_specs=[
            pl.BlockSpec((1, 256, 128), lambda i, c: (c, i, 0)),
            pl.BlockSpec((1, 256, 128), lambda i, c: (4, i, 0)),
            pl.BlockSpec((1, 1, 128), lambda i, c: (c, 0, 0)),
            pl.BlockSpec((1, 128, 128), lambda i, c: (c, 0, 0)),
        ],
        out_specs=pl.BlockSpec((256, 128), lambda i, c: (i, 0)),
        out_shape=jax.ShapeDtypeStruct((NP, 128), jnp.float32),
    )(agg1, agg1, b1r, w2r)


def _fin_body(agg_ref, b_ref, o_ref):
    a = agg_ref[0]
    den = a[:, 66:67] + 1e-16
    o_ref[...] = a[:, :64] / den + b_ref[...][None, :]


def _final(agg2, b2):
    return pl.pallas_call(
        _fin_body,
        grid=(NP // 256,),
        in_specs=[
            pl.BlockSpec((1, 256, 128), lambda i: (0, i, 0)),
            pl.BlockSpec((64,), lambda i: (0,)),
        ],
        out_specs=pl.BlockSpec((256, 64), lambda i: (i, 0)),
        out_shape=jax.ShapeDtypeStruct((NP, 64), jnp.float32),
    )(agg2, b2)


def _route_body(src2_hbm, dst2_hbm, srcR_hbm, dstR_hbm, cnts_hbm,
                srcin, dstin, s0b, d0b, s1b, d1b, cbuf):
    core = lax.axis_index("c")
    sub = lax.axis_index("s")
    wid = sub * 2 + core
    rbase = pl.multiple_of(wid * NB, 8)
    pltpu.sync_copy(src2_hbm.at[pl.ds(rbase, NB)], srcin)
    pltpu.sync_copy(dst2_hbm.at[pl.ds(rbase, NB)], dstin)
    ii = lax.iota(jnp.int32, 16)
    zero_spl = jnp.zeros((16,), jnp.int32)

    def gstep(g, carry):
        c0, c1 = carry
        for l in range(B // 16):
            s16 = srcin[g, pl.ds(l * 16, 16)]
            d16 = dstin[g, pl.ds(l * 16, 16)]
            m0 = d16 < HALF
            m1 = jnp.logical_not(m0)
            pos0 = plsc.cumsum(m0.astype(jnp.int32))
            pos1 = plsc.cumsum(m1.astype(jnp.int32))
            idx0 = c0 + pos0 - 1
            idx1 = c1 + pos1 - 1
            plsc.store_scatter(s0b, [idx0], s16, mask=m0)
            plsc.store_scatter(d0b, [idx0 >> 6, idx0 & 63], d16, mask=m0)
            plsc.store_scatter(s1b, [idx1], s16, mask=m1)
            plsc.store_scatter(d1b, [idx1 >> 6, idx1 & 63], d16, mask=m1)
            c0 = c0 + plsc.all_reduce_population_count(m0)
            c1 = c1 + plsc.all_reduce_population_count(m1)
        return c0, c1

    c0, c1 = lax.fori_loop(0, NB, gstep, (zero_spl, zero_spl))

    nbp0 = jnp.maximum((c0 + 127) // 128, 1)
    nbp1 = jnp.maximum((c1 + 127) // 128, 1)
    dsrc = jnp.zeros((16,), jnp.int32)
    ddst = jnp.full((16,), N_NODES, jnp.int32)
    for j in range(8):
        idx = c0 + j * 16 + ii
        m = idx < nbp0 * 128
        plsc.store_scatter(s0b, [idx], dsrc, mask=m)
        plsc.store_scatter(d0b, [idx >> 6, idx & 63], ddst, mask=m)
        idx = c1 + j * 16 + ii
        m = idx < nbp1 * 128
        plsc.store_scatter(s1b, [idx], dsrc, mask=m)
        plsc.store_scatter(d1b, [idx >> 6, idx & 63], ddst, mask=m)

    cbuf[0] = jnp.where(ii == 0, nbp0 * 2, jnp.where(ii == 1, nbp1 * 2, 0))
    pltpu.sync_copy(cbuf, cnts_hbm.at[wid])
    off0 = pl.multiple_of(wid * SEGCAP, 8)
    off1 = pl.multiple_of((32 + wid) * SEGCAP, 8)
    row0 = pl.multiple_of(wid * SEGB, 8)
    row1 = pl.multiple_of((32 + wid) * SEGB, 8)
    pltpu.sync_copy(s0b, srcR_hbm.at[pl.ds(off0, SEGCAP)])
    pltpu.sync_copy(d0b, dstR_hbm.at[pl.ds(row0, SEGB)])
    pltpu.sync_copy(s1b, srcR_hbm.at[pl.ds(off1, SEGCAP)])
    pltpu.sync_copy(d1b, dstR_hbm.at[pl.ds(row1, SEGB)])


def _make_route():
    mesh = plsc.VectorSubcoreMesh(core_axis_name="c", subcore_axis_name="s")
    return pl.kernel(
        _route_body,
        out_type=(
            jax.ShapeDtypeStruct((2 * 32 * SEGCAP,), jnp.int32),
            jax.ShapeDtypeStruct((2 * 32 * SEGB, B), jnp.int32),
            jax.ShapeDtypeStruct((32, 1, 16), jnp.int32),
        ),
        mesh=mesh,
        compiler_params=pltpu.CompilerParams(needs_layout_passes=False),
        scratch_types=[
            pltpu.VMEM((NB, B), jnp.int32),
            pltpu.VMEM((NB, B), jnp.int32),
            pltpu.VMEM((SEGCAP,), jnp.int32),
            pltpu.VMEM((SEGB, B), jnp.int32),
            pltpu.VMEM((SEGCAP,), jnp.int32),
            pltpu.VMEM((SEGB, B), jnp.int32),
            pltpu.VMEM((1, 16), jnp.int32),
        ],
    )


def _make_agg(ch_count, den_chunk):
    n_out = ch_count + (1 if den_chunk else 0)

    def body(*refs):
        h_refs = refs[:ch_count]
        s_hbm, d_hbm, srcR_hbm, dstR_hbm, cnts_hbm, z_hbm, out_hbm = \
            refs[ch_count:ch_count + 7]
        (s_v, d_v, w_v, src1_v, dst2_v, cbuf2, gbuf, obuf, acc,
         sg0, sg1, ss0, ss1) = refs[ch_count + 7:]

        core = lax.axis_index("c")
        sub = lax.axis_index("s")
        base = core * HALF

        pltpu.sync_copy(s_hbm, s_v)
        pltpu.sync_copy(d_hbm.at[pl.ds(base, HALF)], d_v)
        pltpu.sync_copy(cnts_hbm.at[pl.ds(2 * sub, 2)], cbuf2)

        ii = lax.iota(jnp.int32, 16)
        nbs = [jnp.max(jnp.where(ii == core, cbuf2[k, 0], 0))
               for k in (0, 1)]

        e0 = jnp.where(ii == 0, 1.0, 0.0).astype(jnp.float32)
        zero16 = jnp.zeros((16,), jnp.float32)
        sems_g = (sg0, sg1)
        sems_s = (ss0, ss1)

        def load_seg(seg, nb):
            segoff = pl.multiple_of(
                (core * 32 + 2 * sub + seg) * SEGCAP, 8)
            rowoff = pl.multiple_of(
                (core * 32 + 2 * sub + seg) * SEGB, 8)
            pltpu.sync_copy(srcR_hbm.at[pl.ds(segoff, SEGCAP)], src1_v)
            pltpu.sync_copy(dstR_hbm.at[pl.ds(rowoff, SEGB)], dst2_v)

            def wbody(g, _):
                for l in range(B // 16):
                    off = g * B + l * 16
                    si = src1_v[pl.ds(off, 16)]
                    di = dst2_v[g, pl.ds(l * 16, 16)]
                    dc = di - base
                    ok = (dc >= 0) & (dc < HALF)
                    dloc = jnp.where(ok, dc, 0)
                    sv = plsc.load_gather(s_v, [si])
                    dv = plsc.load_gather(d_v, [dloc])
                    e = sv + dv
                    e = jnp.where(e >= 0.0, e, 0.2 * e)
                    w_v[pl.ds(off, 16)] = jnp.exp(e)
                    dst2_v[g, pl.ds(l * 16, 16)] = jnp.where(ok, dc, HALF)
                return 0

            lax.fori_loop(0, nb, wbody, 0)

        def zero_stripe():
            pltpu.sync_copy(z_hbm.at[pl.ds(0, STRIPE)],
                            acc.at[pl.ds(sub * STRIPE, STRIPE)])
            plsc.subcore_barrier()

        def drain_stripe(ch):
            off = pl.multiple_of(ch * NP + base + sub * STRIPE, 8)
            pltpu.sync_copy(acc.at[pl.ds(sub * STRIPE, STRIPE)],
                            out_hbm.at[pl.ds(off, STRIPE)])
            plsc.subcore_barrier()

        def run_seg(nb, h_hbm):
            def gather_batch(g, b):
                pltpu.async_copy(
                    h_hbm.at[src1_v.at[pl.ds(pl.multiple_of(g * B, 64), B)]],
                    gbuf.at[pl.ds(b * B, B)], sems_g[b])

            if h_hbm is not None:
                for b in (0, 1):
                    gather_batch(jnp.int32(b), b)

            def batch_pair(g2, _):
                for b in (0, 1):
                    g = 2 * g2 + b
                    if h_hbm is not None:
                        pltpu.make_async_copy(
                            h_hbm.at[src1_v.at[pl.ds(
                                pl.multiple_of(g * B, 64), B)]],
                            gbuf.at[pl.ds(b * B, B)], sems_g[b]).wait()

                    @pl.when(g >= 2)
                    def _():
                        pltpu.make_async_copy(
                            obuf.at[pl.ds(b * B, B)],
                            acc.at[dst2_v.at[g - 2]],
                            sems_s[b]).wait()

                    gB = g * B

                    if h_hbm is not None:
                        @plsc.parallel_loop(0, B, step=1, unroll=4)
                        def row(i):
                            widx = jnp.broadcast_to(
                                (gB + i).astype(jnp.int32), (16,))
                            ws = plsc.load_gather(w_v, [widx])
                            rr = b * B + i
                            for k in range(8):
                                v = gbuf[rr, pl.ds(k * 16, 16)]
                                obuf[rr, pl.ds(k * 16, 16)] = v * ws
                    else:
                        @plsc.parallel_loop(0, B, step=1, unroll=4)
                        def drow(i):
                            widx = jnp.broadcast_to(
                                (gB + i).astype(jnp.int32), (16,))
                            ws = plsc.load_gather(w_v, [widx])
                            obuf[b * B + i, pl.ds(0, 16)] = ws * e0

                    pltpu.async_copy(
                        obuf.at[pl.ds(b * B, B)],
                        acc.at[dst2_v.at[g]], sems_s[b], add=True)

                    if h_hbm is not None:
                        @pl.when(g + 2 < nb)
                        def _():
                            gather_batch(g + 2, b)
                return 0

            lax.fori_loop(0, nb // 2, batch_pair, 0)

            for b in (0, 1):
                pltpu.make_async_copy(
                    obuf.at[pl.ds(b * B, B)],
                    acc.at[dst2_v.at[nb - 2 + b]], sems_s[b]).wait()

        for ch in range(ch_count):
            zero_stripe()
            for seg in (0, 1):
                load_seg(seg, nbs[seg])
                run_seg(nbs[seg], h_refs[ch])
            plsc.subcore_barrier()
            drain_stripe(ch)

        if den_chunk:
            zero_stripe()

            @plsc.parallel_loop(0, 2 * B, step=1, unroll=2)
            def zrow(i):
                for k in range(1, 8):
                    obuf[i, pl.ds(k * 16, 16)] = zero16

            for seg in (0, 1):
                load_seg(seg, nbs[seg])
                run_seg(nbs[seg], None)
            plsc.subcore_barrier()
            drain_stripe(ch_count)

    mesh = plsc.VectorSubcoreMesh(core_axis_name="c", subcore_axis_name="s")
    return pl.kernel(
        body,
        out_type=jax.ShapeDtypeStruct((n_out * NP, 128), jnp.float32),
        mesh=mesh,
        compiler_params=pltpu.CompilerParams(needs_layout_passes=False),
        scratch_types=[
            pltpu.VMEM((NP,), jnp.float32),
            pltpu.VMEM((HALF,), jnp.float32),
            pltpu.VMEM((SEGCAP,), jnp.float32),
            pltpu.VMEM((SEGCAP,), jnp.int32),
            pltpu.VMEM((SEGB, B), jnp.int32),
            pltpu.VMEM((2, 1, 16), jnp.int32),
            pltpu.VMEM((2 * B, 128), jnp.float32),
            pltpu.VMEM((2 * B, 128), jnp.float32),
            pltpu.VMEM_SHARED((HALF + 16, 128), jnp.float32),
            pltpu.SemaphoreType.DMA,
            pltpu.SemaphoreType.DMA,
            pltpu.SemaphoreType.DMA,
            pltpu.SemaphoreType.DMA,
        ],
    )


@jax.jit
def _run(x, src2, dst2, W1ext, b1r, W2ext3, b2):
    zeros = jnp.zeros((STRIPE, 128), jnp.float32)

    srcR, dstR, cnts = _make_route()(src2, dst2)

    h1 = _matmul_chunks(x, W1ext, 5)
    s1 = h1[4, :, 0]
    d1 = h1[4, :, 1]
    agg1_fn = _make_agg(4, True)
    agg1 = agg1_fn(h1[0], h1[1], h1[2], h1[3], s1, d1, srcR, dstR, cnts,
                   zeros)
    agg1 = agg1.reshape(5, NP, 128)

    out2 = _matmul2(agg1, b1r, W2ext3)
    s2 = out2[:, 64]
    d2 = out2[:, 65]
    agg2_fn = _make_agg(1, False)
    agg2 = agg2_fn(out2, s2, d2, srcR, dstR, cnts, zeros)
    agg2 = agg2.reshape(1, NP, 128)

    out = _final(agg2, b2)
    return out[:N_NODES]


def kernel(x, edge_index, W1, a1_src, a1_dst, b1, W2, a2_src, a2_dst, b2):
    ei = edge_index.astype(jnp.int32)
    loop = jnp.arange(N_NODES, dtype=jnp.int32)
    src = jnp.concatenate([ei[0], loop])
    dst = jnp.concatenate([ei[1], loop])
    src2 = jnp.pad(src, (0, EP - EL)).reshape(EP // B, B)
    dst2 = jnp.pad(dst, (0, EP - EL),
                   constant_values=N_NODES).reshape(EP // B, B)

    xp = jnp.pad(x, ((0, NP - N_NODES), (0, 0)))

    W1ext = jnp.zeros((256, 640), jnp.float32)
    W1ext = W1ext.at[:, :512].set(W1)
    W1ext = W1ext.at[:, 512].set(W1 @ a1_src)
    W1ext = W1ext.at[:, 513].set(W1 @ a1_dst)

    W2ext = jnp.zeros((512, 128), jnp.float32)
    W2ext = W2ext.at[:, :64].set(W2)
    W2ext = W2ext.at[:, 64].set(W2 @ a2_src)
    W2ext = W2ext.at[:, 65].set(W2 @ a2_dst)
    W2ext3 = W2ext.reshape(4, 128, 128)

    b1r = b1.reshape(4, 1, 128)

    return _run(xp, src2, dst2, W1ext, b1r, W2ext3, b2)


# speedup vs baseline: 3.3107x; 1.0195x over previous
import jax
import jax.numpy as jnp
from jax import lax
from jax.experimental import pallas as pl
from jax.experimental.pallas import tpu as pltpu
from jax.experimental.pallas import tpu_sc as plsc

N_NODES = 10000
NP = 10240
HALF = NP // 2
E_RAW = 160000
EL = E_RAW + N_NODES
B = 64
EP = 180224
ET = EP // 32
NB = ET // B
SEGB = 96
SEGCAP = SEGB * B
STRIPE = HALF // 16


def _mm_body(x_ref, w_ref, o_ref):
    o_ref[0] = jnp.dot(x_ref[...], w_ref[...],
                       preferred_element_type=jnp.float32)


def _matmul_chunks(x, w, n_chunks):
    m, k = x.shape
    grid = (m // 256, n_chunks)
    return pl.pallas_call(
        _mm_body,
        grid=grid,
        in_specs=[
            pl.BlockSpec((256, k), lambda i, c: (i, 0)),
            pl.BlockSpec((k, 128), lambda i, c: (0, c)),
        ],
        out_specs=pl.BlockSpec((1, 256, 128), lambda i, c: (c, i, 0)),
        out_shape=jax.ShapeDtypeStruct((n_chunks, m, 128), jnp.float32),
    )(x, w)


def _mm2_body(agg_ref, den_ref, b_ref, w_ref, o_ref):
    c = pl.program_id(1)
    a = agg_ref[0]
    den = den_ref[0, 0][:, None] + 1e-16
    h = a / den + b_ref[0, 0][None, :]
    h = jnp.maximum(h, 0.0)
    contrib = jnp.dot(h, w_ref[0], preferred_element_type=jnp.float32)

    @pl.when(c == 0)
    def _():
        col = lax.broadcasted_iota(jnp.int32, (256, 128), 1)
        o_ref[...] = contrib + jnp.where(col == 66, 1.0, 0.0)

    @pl.when(c > 0)
    def _():
        o_ref[...] += contrib


def _matmul2(agg1, den1r, b1r, w2r):
    grid = (NP // 256, 4)
    return pl.pallas_call(
        _mm2_body,
        grid=grid,
        in_specs=[
            pl.BlockSpec((1, 256, 128), lambda i, c: (c, i, 0)),
            pl.BlockSpec((1, 1, 256), lambda i, c: (i, 0, 0)),
            pl.BlockSpec((1, 1, 128), lambda i, c: (c, 0, 0)),
            pl.BlockSpec((1, 128, 128), lambda i, c: (c, 0, 0)),
        ],
        out_specs=pl.BlockSpec((256, 128), lambda i, c: (i, 0)),
        out_shape=jax.ShapeDtypeStruct((NP, 128), jnp.float32),
    )(agg1, den1r, b1r, w2r)


def _fin_body(agg_ref, b_ref, o_ref):
    a = agg_ref[0]
    den = a[:, 66:67] + 1e-16
    o_ref[...] = a[:, :64] / den + b_ref[...][None, :]


def _final(agg2, b2):
    return pl.pallas_call(
        _fin_body,
        grid=(NP // 256,),
        in_specs=[
            pl.BlockSpec((1, 256, 128), lambda i: (0, i, 0)),
            pl.BlockSpec((64,), lambda i: (0,)),
        ],
        out_specs=pl.BlockSpec((256, 64), lambda i: (i, 0)),
        out_shape=jax.ShapeDtypeStruct((NP, 64), jnp.float32),
    )(agg2, b2)


def _route_body(src2_hbm, dst2_hbm, srcR_hbm, dstR_hbm, cnts_hbm,
                srcin, dstin, s0b, d0b, s1b, d1b, cbuf):
    core = lax.axis_index("c")
    sub = lax.axis_index("s")
    wid = sub * 2 + core
    rbase = pl.multiple_of(wid * NB, 8)
    pltpu.sync_copy(src2_hbm.at[pl.ds(rbase, NB)], srcin)
    pltpu.sync_copy(dst2_hbm.at[pl.ds(rbase, NB)], dstin)
    ii = lax.iota(jnp.int32, 16)
    zero_spl = jnp.zeros((16,), jnp.int32)

    def gstep(g, carry):
        c0, c1 = carry
        for l in range(B // 16):
            s16 = srcin[g, pl.ds(l * 16, 16)]
            d16 = dstin[g, pl.ds(l * 16, 16)]
            m0 = d16 < HALF
            m1 = jnp.logical_not(m0)
            pos0 = plsc.cumsum(m0.astype(jnp.int32))
            pos1 = plsc.cumsum(m1.astype(jnp.int32))
            idx0 = c0 + pos0 - 1
            idx1 = c1 + pos1 - 1
            plsc.store_scatter(s0b, [idx0], s16, mask=m0)
            plsc.store_scatter(d0b, [idx0 >> 6, idx0 & 63], d16, mask=m0)
            plsc.store_scatter(s1b, [idx1], s16, mask=m1)
            plsc.store_scatter(d1b, [idx1 >> 6, idx1 & 63], d16, mask=m1)
            c0 = c0 + plsc.all_reduce_population_count(m0)
            c1 = c1 + plsc.all_reduce_population_count(m1)
        return c0, c1

    c0, c1 = lax.fori_loop(0, NB, gstep, (zero_spl, zero_spl))

    nbp0 = jnp.maximum((c0 + 127) // 128, 1)
    nbp1 = jnp.maximum((c1 + 127) // 128, 1)
    dsrc = jnp.zeros((16,), jnp.int32)
    ddst = jnp.full((16,), N_NODES, jnp.int32)
    for j in range(8):
        idx = c0 + j * 16 + ii
        m = idx < nbp0 * 128
        plsc.store_scatter(s0b, [idx], dsrc, mask=m)
        plsc.store_scatter(d0b, [idx >> 6, idx & 63], ddst, mask=m)
        idx = c1 + j * 16 + ii
        m = idx < nbp1 * 128
        plsc.store_scatter(s1b, [idx], dsrc, mask=m)
        plsc.store_scatter(d1b, [idx >> 6, idx & 63], ddst, mask=m)

    cbuf[0] = jnp.where(ii == 0, nbp0 * 2, jnp.where(ii == 1, nbp1 * 2, 0))
    pltpu.sync_copy(cbuf, cnts_hbm.at[wid])
    off0 = pl.multiple_of(wid * SEGCAP, 8)
    off1 = pl.multiple_of((32 + wid) * SEGCAP, 8)
    row0 = pl.multiple_of(wid * SEGB, 8)
    row1 = pl.multiple_of((32 + wid) * SEGB, 8)
    pltpu.sync_copy(s0b, srcR_hbm.at[pl.ds(off0, SEGCAP)])
    pltpu.sync_copy(d0b, dstR_hbm.at[pl.ds(row0, SEGB)])
    pltpu.sync_copy(s1b, srcR_hbm.at[pl.ds(off1, SEGCAP)])
    pltpu.sync_copy(d1b, dstR_hbm.at[pl.ds(row1, SEGB)])


def _make_route():
    mesh = plsc.VectorSubcoreMesh(core_axis_name="c", subcore_axis_name="s")
    return pl.kernel(
        _route_body,
        out_type=(
            jax.ShapeDtypeStruct((2 * 32 * SEGCAP,), jnp.int32),
            jax.ShapeDtypeStruct((2 * 32 * SEGB, B), jnp.int32),
            jax.ShapeDtypeStruct((32, 1, 16), jnp.int32),
        ),
        mesh=mesh,
        compiler_params=pltpu.CompilerParams(needs_layout_passes=False),
        scratch_types=[
            pltpu.VMEM((NB, B), jnp.int32),
            pltpu.VMEM((NB, B), jnp.int32),
            pltpu.VMEM((SEGCAP,), jnp.int32),
            pltpu.VMEM((SEGB, B), jnp.int32),
            pltpu.VMEM((SEGCAP,), jnp.int32),
            pltpu.VMEM((SEGB, B), jnp.int32),
            pltpu.VMEM((1, 16), jnp.int32),
        ],
    )


def _make_agg(ch_count, den_mode):
    n_out = ch_count

    def body(*refs):
        h_refs = refs[:ch_count]
        s_hbm, d_hbm, srcR_hbm, dstR_hbm, cnts_hbm, z_hbm = \
            refs[ch_count:ch_count + 6]
        if den_mode:
            out_hbm, den_hbm = refs[ch_count + 6:ch_count + 8]
            (s_v, d_v, w_v, src1_v, dst2_v, cbuf2, gbuf, obuf, den_t,
             acc, den_stage, sg0, sg1, ss0, ss1) = refs[ch_count + 8:]
        else:
            out_hbm = refs[ch_count + 6]
            (s_v, d_v, w_v, src1_v, dst2_v, cbuf2, gbuf, obuf, acc,
             sg0, sg1, ss0, ss1) = refs[ch_count + 7:]

        core = lax.axis_index("c")
        sub = lax.axis_index("s")
        base = core * HALF

        pltpu.sync_copy(s_hbm, s_v)
        pltpu.sync_copy(d_hbm.at[pl.ds(base, HALF)], d_v)
        pltpu.sync_copy(cnts_hbm.at[pl.ds(2 * sub, 2)], cbuf2)

        ii = lax.iota(jnp.int32, 16)
        nbs = [jnp.max(jnp.where(ii == core, cbuf2[k, 0], 0))
               for k in (0, 1)]

        zero16 = jnp.zeros((16,), jnp.float32)
        sems_g = (sg0, sg1)
        sems_s = (ss0, ss1)

        if den_mode:
            @plsc.parallel_loop(0, 5136 // 16, step=1, unroll=2)
            def zden(q):
                den_t[pl.ds(q * 16, 16)] = zero16

        def load_seg(seg, nb, do_den):
            segoff = pl.multiple_of(
                (core * 32 + 2 * sub + seg) * SEGCAP, 8)
            rowoff = pl.multiple_of(
                (core * 32 + 2 * sub + seg) * SEGB, 8)
            pltpu.sync_copy(srcR_hbm.at[pl.ds(segoff, SEGCAP)], src1_v)
            pltpu.sync_copy(dstR_hbm.at[pl.ds(rowoff, SEGB)], dst2_v)

            def wbody(g, _):
                for l in range(B // 16):
                    off = g * B + l * 16
                    si = src1_v[pl.ds(off, 16)]
                    di = dst2_v[g, pl.ds(l * 16, 16)]
                    dc = di - base
                    ok = (dc >= 0) & (dc < HALF)
                    dloc = jnp.where(ok, dc, 0)
                    sv = plsc.load_gather(s_v, [si])
                    dv = plsc.load_gather(d_v, [dloc])
                    e = sv + dv
                    e = jnp.where(e >= 0.0, e, 0.2 * e)
                    w = jnp.exp(e)
                    w_v[pl.ds(off, 16)] = w
                    dfin = jnp.where(ok, dc, HALF)
                    dst2_v[g, pl.ds(l * 16, 16)] = dfin
                    if do_den:
                        plsc.addupdate_scatter(den_t, [dfin], w)
                return 0

            lax.fori_loop(0, nb, wbody, 0)

        def zero_stripe():
            pltpu.sync_copy(z_hbm.at[pl.ds(0, STRIPE)],
                            acc.at[pl.ds(sub * STRIPE, STRIPE)])
            plsc.subcore_barrier()

        def drain_stripe(ch):
            off = pl.multiple_of(ch * NP + base + sub * STRIPE, 8)
            pltpu.sync_copy(acc.at[pl.ds(sub * STRIPE, STRIPE)],
                            out_hbm.at[pl.ds(off, STRIPE)])
            plsc.subcore_barrier()

        def run_seg(nb, h_hbm):
            def gather_batch(g, b):
                pltpu.async_copy(
                    h_hbm.at[src1_v.at[pl.ds(pl.multiple_of(g * B, 64), B)]],
                    gbuf.at[pl.ds(b * B, B)], sems_g[b])

            for b in (0, 1):
                gather_batch(jnp.int32(b), b)

            def batch_pair(g2, _):
                for b in (0, 1):
                    g = 2 * g2 + b
                    pltpu.make_async_copy(
                        h_hbm.at[src1_v.at[pl.ds(
                            pl.multiple_of(g * B, 64), B)]],
                        gbuf.at[pl.ds(b * B, B)], sems_g[b]).wait()

                    @pl.when(g >= 2)
                    def _():
                        pltpu.make_async_copy(
                            obuf.at[pl.ds(b * B, B)],
                            acc.at[dst2_v.at[g - 2]],
                            sems_s[b]).wait()

                    gB = g * B

                    @plsc.parallel_loop(0, B, step=1, unroll=4)
                    def row(i):
                        widx = jnp.broadcast_to(
                            (gB + i).astype(jnp.int32), (16,))
                        ws = plsc.load_gather(w_v, [widx])
                        rr = b * B + i
                        for k in range(8):
                            v = gbuf[rr, pl.ds(k * 16, 16)]
                            obuf[rr, pl.ds(k * 16, 16)] = v * ws

                    pltpu.async_copy(
                        obuf.at[pl.ds(b * B, B)],
                        acc.at[dst2_v.at[g]], sems_s[b], add=True)

                    @pl.when(g + 2 < nb)
                    def _():
                        gather_batch(g + 2, b)
                return 0

            lax.fori_loop(0, nb // 2, batch_pair, 0)

            for b in (0, 1):
                pltpu.make_async_copy(
                    obuf.at[pl.ds(b * B, B)],
                    acc.at[dst2_v.at[nb - 2 + b]], sems_s[b]).wait()

        for ch in range(ch_count):
            zero_stripe()
            for seg in (0, 1):
                load_seg(seg, nbs[seg], den_mode and ch == 0)
                run_seg(nbs[seg], h_refs[ch])
            plsc.subcore_barrier()
            drain_stripe(ch)

        if den_mode:
            pltpu.sync_copy(den_t.at[pl.ds(0, HALF)],
                            den_stage.at[sub, 0, pl.ds(0, HALF)])
            plsc.subcore_barrier()
            cbase = pl.multiple_of(jnp.minimum(sub, 13) * 384, 128)
            for rnd in (0, 1):
                for j in range(8):
                    pltpu.sync_copy(
                        den_stage.at[rnd * 8 + j, 0, pl.ds(cbase, 384)],
                        den_t.at[pl.ds(j * 384, 384)])

                @plsc.parallel_loop(0, 384 // 16, step=1, unroll=2)
                def dred(q):
                    tot = den_t[pl.ds(q * 16, 16)]
                    for j in range(1, 8):
                        tot = tot + den_t[pl.ds(j * 384 + q * 16, 16)]
                    if rnd == 0:
                        den_t[pl.ds(3072 + q * 16, 16)] = tot
                    else:
                        den_t[pl.ds(3072 + q * 16, 16)] = (
                            tot + den_t[pl.ds(3072 + q * 16, 16)])

            @pl.when(sub < 13)
            def _():
                pltpu.sync_copy(
                    den_t.at[pl.ds(3072, 384)],
                    den_hbm.at[pl.ds(
                        pl.multiple_of(base + cbase, 8), 384)])

            @pl.when(sub == 13)
            def _():
                pltpu.sync_copy(
                    den_t.at[pl.ds(3072, 128)],
                    den_hbm.at[pl.ds(
                        pl.multiple_of(base + 4992, 8), 128)])

    mesh = plsc.VectorSubcoreMesh(core_axis_name="c", subcore_axis_name="s")
    if den_mode:
        out_type = (
            jax.ShapeDtypeStruct((n_out * NP, 128), jnp.float32),
            jax.ShapeDtypeStruct((NP,), jnp.float32),
        )
        extra = [pltpu.VMEM((HALF + 16,), jnp.float32)]
        extra_sh = [pltpu.VMEM_SHARED((16, 1, 5376), jnp.float32)]
    else:
        out_type = jax.ShapeDtypeStruct((n_out * NP, 128), jnp.float32)
        extra = []
        extra_sh = []
    return pl.kernel(
        body,
        out_type=out_type,
        mesh=mesh,
        compiler_params=pltpu.CompilerParams(needs_layout_passes=False),
        scratch_types=[
            pltpu.VMEM((NP,), jnp.float32),
            pltpu.VMEM((HALF,), jnp.float32),
            pltpu.VMEM((SEGCAP,), jnp.float32),
            pltpu.VMEM((SEGCAP,), jnp.int32),
            pltpu.VMEM((SEGB, B), jnp.int32),
            pltpu.VMEM((2, 1, 16), jnp.int32),
            pltpu.VMEM((2 * B, 128), jnp.float32),
            pltpu.VMEM((2 * B, 128), jnp.float32),
        ] + extra + [
            pltpu.VMEM_SHARED((HALF + 16, 128), jnp.float32),
        ] + extra_sh + [
            pltpu.SemaphoreType.DMA,
            pltpu.SemaphoreType.DMA,
            pltpu.SemaphoreType.DMA,
            pltpu.SemaphoreType.DMA,
        ],
    )


@jax.jit
def _run(x, src2, dst2, W1ext, b1r, W2ext3, b2):
    zeros = jnp.zeros((STRIPE, 128), jnp.float32)

    srcR, dstR, cnts = _make_route()(src2, dst2)

    h1 = _matmul_chunks(x, W1ext, 5)
    s1 = h1[4, :, 0]
    d1 = h1[4, :, 1]
    agg1_fn = _make_agg(4, True)
    agg1, den1 = agg1_fn(h1[0], h1[1], h1[2], h1[3], s1, d1, srcR, dstR,
                         cnts, zeros)
    agg1 = agg1.reshape(4, NP, 128)
    den1r = den1.reshape(NP // 256, 1, 256)

    out2 = _matmul2(agg1, den1r, b1r, W2ext3)
    s2 = out2[:, 64]
    d2 = out2[:, 65]
    agg2_fn = _make_agg(1, False)
    agg2 = agg2_fn(out2, s2, d2, srcR, dstR, cnts, zeros)
    agg2 = agg2.reshape(1, NP, 128)

    out = _final(agg2, b2)
    return out[:N_NODES]


def kernel(x, edge_index, W1, a1_src, a1_dst, b1, W2, a2_src, a2_dst, b2):
    ei = edge_index.astype(jnp.int32)
    loop = jnp.arange(N_NODES, dtype=jnp.int32)
    src = jnp.concatenate([ei[0], loop])
    dst = jnp.concatenate([ei[1], loop])
    src2 = jnp.pad(src, (0, EP - EL)).reshape(EP // B, B)
    dst2 = jnp.pad(dst, (0, EP - EL),
                   constant_values=N_NODES).reshape(EP // B, B)

    xp = jnp.pad(x, ((0, NP - N_NODES), (0, 0)))

    W1ext = jnp.zeros((256, 640), jnp.float32)
    W1ext = W1ext.at[:, :512].set(W1)
    W1ext = W1ext.at[:, 512].set(W1 @ a1_src)
    W1ext = W1ext.at[:, 513].set(W1 @ a1_dst)

    W2ext = jnp.zeros((512, 128), jnp.float32)
    W2ext = W2ext.at[:, :64].set(W2)
    W2ext = W2ext.at[:, 64].set(W2 @ a2_src)
    W2ext = W2ext.at[:, 65].set(W2 @ a2_dst)
    W2ext3 = W2ext.reshape(4, 128, 128)

    b1r = b1.reshape(4, 1, 128)

    return _run(xp, src2, dst2, W1ext, b1r, W2ext3, b2)


# speedup vs baseline: 3.3133x; 1.0008x over previous
import jax
import jax.numpy as jnp
from jax import lax
from jax.experimental import pallas as pl
from jax.experimental.pallas import tpu as pltpu
from jax.experimental.pallas import tpu_sc as plsc

N_NODES = 10000
NP = 10240
HALF = NP // 2
E_RAW = 160000
EL = E_RAW + N_NODES
B = 64
EP = 180224
ET = EP // 32
NB = ET // B
SEGB = 96
SEGCAP = SEGB * B
STRIPE = HALF // 16


def _mm_body(x_ref, w_ref, o_ref):
    o_ref[0] = jnp.dot(x_ref[...], w_ref[...],
                       preferred_element_type=jnp.float32)


def _matmul_chunks(x, w, n_chunks):
    m, k = x.shape
    grid = (m // 256, n_chunks)
    return pl.pallas_call(
        _mm_body,
        grid=grid,
        in_specs=[
            pl.BlockSpec((256, k), lambda i, c: (i, 0)),
            pl.BlockSpec((k, 128), lambda i, c: (0, c)),
        ],
        out_specs=pl.BlockSpec((1, 256, 128), lambda i, c: (c, i, 0)),
        out_shape=jax.ShapeDtypeStruct((n_chunks, m, 128), jnp.float32),
    )(x, w)


def _mm2_body(agg_ref, den_ref, b_ref, w_ref, o_ref):
    c = pl.program_id(1)
    a = agg_ref[0]
    den = den_ref[0, 0][:, None] + 1e-16
    h = a / den + b_ref[0, 0][None, :]
    h = jnp.maximum(h, 0.0)
    contrib = jnp.dot(h, w_ref[0], preferred_element_type=jnp.float32)

    @pl.when(c == 0)
    def _():
        col = lax.broadcasted_iota(jnp.int32, (256, 128), 1)
        o_ref[...] = contrib + jnp.where(col == 66, 1.0, 0.0)

    @pl.when(c > 0)
    def _():
        o_ref[...] += contrib


def _matmul2(agg1, den1r, b1r, w2r):
    grid = (NP // 256, 4)
    return pl.pallas_call(
        _mm2_body,
        grid=grid,
        in_specs=[
            pl.BlockSpec((1, 256, 128), lambda i, c: (c, i, 0)),
            pl.BlockSpec((1, 1, 256), lambda i, c: (i, 0, 0)),
            pl.BlockSpec((1, 1, 128), lambda i, c: (c, 0, 0)),
            pl.BlockSpec((1, 128, 128), lambda i, c: (c, 0, 0)),
        ],
        out_specs=pl.BlockSpec((256, 128), lambda i, c: (i, 0)),
        out_shape=jax.ShapeDtypeStruct((NP, 128), jnp.float32),
    )(agg1, den1r, b1r, w2r)


def _fin_body(agg_ref, b_ref, o_ref):
    a = agg_ref[0]
    den = a[:, 66:67] + 1e-16
    o_ref[...] = a[:, :64] / den + b_ref[...][None, :]


def _final(agg2, b2):
    return pl.pallas_call(
        _fin_body,
        grid=(NP // 256,),
        in_specs=[
            pl.BlockSpec((1, 256, 128), lambda i: (0, i, 0)),
            pl.BlockSpec((64,), lambda i: (0,)),
        ],
        out_specs=pl.BlockSpec((256, 64), lambda i: (i, 0)),
        out_shape=jax.ShapeDtypeStruct((NP, 64), jnp.float32),
    )(agg2, b2)


def _route_body(src2_hbm, dst2_hbm, srcR_hbm, dstR_hbm, cnts_hbm,
                srcin, dstin, s0b, d0b, s1b, d1b, cbuf):
    core = lax.axis_index("c")
    sub = lax.axis_index("s")
    wid = sub * 2 + core
    rbase = pl.multiple_of(wid * NB, 8)
    pltpu.sync_copy(src2_hbm.at[pl.ds(rbase, NB)], srcin)
    pltpu.sync_copy(dst2_hbm.at[pl.ds(rbase, NB)], dstin)
    ii = lax.iota(jnp.int32, 16)
    zero_spl = jnp.zeros((16,), jnp.int32)

    def gstep(g, carry):
        c0, c1 = carry
        for l in range(B // 16):
            s16 = srcin[g, pl.ds(l * 16, 16)]
            d16 = dstin[g, pl.ds(l * 16, 16)]
            m0 = d16 < HALF
            m1 = jnp.logical_not(m0)
            pos0 = plsc.cumsum(m0.astype(jnp.int32))
            pos1 = plsc.cumsum(m1.astype(jnp.int32))
            idx0 = c0 + pos0 - 1
            idx1 = c1 + pos1 - 1
            plsc.store_scatter(s0b, [idx0], s16, mask=m0)
            plsc.store_scatter(d0b, [idx0 >> 6, idx0 & 63], d16, mask=m0)
            plsc.store_scatter(s1b, [idx1], s16, mask=m1)
            plsc.store_scatter(d1b, [idx1 >> 6, idx1 & 63], d16, mask=m1)
            c0 = c0 + plsc.all_reduce_population_count(m0)
            c1 = c1 + plsc.all_reduce_population_count(m1)
        return c0, c1

    c0, c1 = lax.fori_loop(0, NB, gstep, (zero_spl, zero_spl))

    nbp0 = jnp.maximum((c0 + 127) // 128, 1)
    nbp1 = jnp.maximum((c1 + 127) // 128, 1)
    dsrc = jnp.zeros((16,), jnp.int32)
    for j in range(8):
        ddst = N_NODES + j * 16 + ii
        idx = c0 + j * 16 + ii
        m = idx < nbp0 * 128
        plsc.store_scatter(s0b, [idx], dsrc, mask=m)
        plsc.store_scatter(d0b, [idx >> 6, idx & 63], ddst, mask=m)
        idx = c1 + j * 16 + ii
        m = idx < nbp1 * 128
        plsc.store_scatter(s1b, [idx], dsrc, mask=m)
        plsc.store_scatter(d1b, [idx >> 6, idx & 63], ddst, mask=m)

    cbuf[0] = jnp.where(ii == 0, nbp0 * 2, jnp.where(ii == 1, nbp1 * 2, 0))
    pltpu.sync_copy(cbuf, cnts_hbm.at[wid])
    off0 = pl.multiple_of(wid * SEGCAP, 8)
    off1 = pl.multiple_of((32 + wid) * SEGCAP, 8)
    row0 = pl.multiple_of(wid * SEGB, 8)
    row1 = pl.multiple_of((32 + wid) * SEGB, 8)
    pltpu.sync_copy(s0b, srcR_hbm.at[pl.ds(off0, SEGCAP)])
    pltpu.sync_copy(d0b, dstR_hbm.at[pl.ds(row0, SEGB)])
    pltpu.sync_copy(s1b, srcR_hbm.at[pl.ds(off1, SEGCAP)])
    pltpu.sync_copy(d1b, dstR_hbm.at[pl.ds(row1, SEGB)])


def _make_route():
    mesh = plsc.VectorSubcoreMesh(core_axis_name="c", subcore_axis_name="s")
    return pl.kernel(
        _route_body,
        out_type=(
            jax.ShapeDtypeStruct((2 * 32 * SEGCAP,), jnp.int32),
            jax.ShapeDtypeStruct((2 * 32 * SEGB, B), jnp.int32),
            jax.ShapeDtypeStruct((32, 1, 16), jnp.int32),
        ),
        mesh=mesh,
        compiler_params=pltpu.CompilerParams(needs_layout_passes=False),
        scratch_types=[
            pltpu.VMEM((NB, B), jnp.int32),
            pltpu.VMEM((NB, B), jnp.int32),
            pltpu.VMEM((SEGCAP,), jnp.int32),
            pltpu.VMEM((SEGB, B), jnp.int32),
            pltpu.VMEM((SEGCAP,), jnp.int32),
            pltpu.VMEM((SEGB, B), jnp.int32),
            pltpu.VMEM((1, 16), jnp.int32),
        ],
    )


def _make_agg(ch_count, den_mode):
    n_out = ch_count

    def body(*refs):
        h_refs = refs[:ch_count]
        s_hbm, d_hbm, srcR_hbm, dstR_hbm, cnts_hbm, z_hbm = \
            refs[ch_count:ch_count + 6]
        if den_mode:
            out_hbm, den_hbm = refs[ch_count + 6:ch_count + 8]
            (s_v, d_v, w_v, src1_v, dst2_v, cbuf2, gbuf, obuf, den_t,
             acc, den_stage, sg0, sg1, ss0, ss1) = refs[ch_count + 8:]
        else:
            out_hbm = refs[ch_count + 6]
            (s_v, d_v, w_v, src1_v, dst2_v, cbuf2, gbuf, obuf, acc,
             sg0, sg1, ss0, ss1) = refs[ch_count + 7:]

        core = lax.axis_index("c")
        sub = lax.axis_index("s")
        base = core * HALF

        pltpu.sync_copy(s_hbm, s_v)
        pltpu.sync_copy(d_hbm.at[pl.ds(base, HALF)], d_v)
        pltpu.sync_copy(cnts_hbm.at[pl.ds(2 * sub, 2)], cbuf2)

        ii = lax.iota(jnp.int32, 16)
        nbs = [jnp.max(jnp.where(ii == core, cbuf2[k, 0], 0))
               for k in (0, 1)]

        zero16 = jnp.zeros((16,), jnp.float32)
        sems_g = (sg0, sg1)
        sems_s = (ss0, ss1)

        if den_mode:
            @plsc.parallel_loop(0, 5136 // 16, step=1, unroll=2)
            def zden(q):
                den_t[pl.ds(q * 16, 16)] = zero16

        def load_seg(seg, nb, do_den):
            segoff = pl.multiple_of(
                (core * 32 + 2 * sub + seg) * SEGCAP, 8)
            rowoff = pl.multiple_of(
                (core * 32 + 2 * sub + seg) * SEGB, 8)
            pltpu.sync_copy(srcR_hbm.at[pl.ds(segoff, SEGCAP)], src1_v)
            pltpu.sync_copy(dstR_hbm.at[pl.ds(rowoff, SEGB)], dst2_v)

            def wbody(g, _):
                for l in range(B // 16):
                    off = g * B + l * 16
                    si = src1_v[pl.ds(off, 16)]
                    di = dst2_v[g, pl.ds(l * 16, 16)]
                    dc = di - base
                    ok = (dc >= 0) & (dc < HALF)
                    dloc = jnp.where(ok, dc, 0)
                    sv = plsc.load_gather(s_v, [si])
                    dv = plsc.load_gather(d_v, [dloc])
                    e = sv + dv
                    e = jnp.where(e >= 0.0, e, 0.2 * e)
                    w = jnp.exp(e)
                    w_v[pl.ds(off, 16)] = w
                    dfin = jnp.where(ok, dc, HALF + (di & 15))
                    dst2_v[g, pl.ds(l * 16, 16)] = dfin
                    if do_den:
                        plsc.addupdate_scatter(den_t, [dfin], w)
                return 0

            lax.fori_loop(0, nb, wbody, 0)

        def zero_stripe():
            pltpu.sync_copy(z_hbm.at[pl.ds(0, STRIPE)],
                            acc.at[pl.ds(sub * STRIPE, STRIPE)])
            plsc.subcore_barrier()

        def drain_stripe(ch):
            off = pl.multiple_of(ch * NP + base + sub * STRIPE, 8)
            pltpu.sync_copy(acc.at[pl.ds(sub * STRIPE, STRIPE)],
                            out_hbm.at[pl.ds(off, STRIPE)])
            plsc.subcore_barrier()

        def run_seg(nb, h_hbm):
            def gather_batch(g, b):
                pltpu.async_copy(
                    h_hbm.at[src1_v.at[pl.ds(pl.multiple_of(g * B, 64), B)]],
                    gbuf.at[pl.ds(b * B, B)], sems_g[b])

            for b in (0, 1):
                gather_batch(jnp.int32(b), b)

            def batch_pair(g2, _):
                for b in (0, 1):
                    g = 2 * g2 + b
                    pltpu.make_async_copy(
                        h_hbm.at[src1_v.at[pl.ds(
                            pl.multiple_of(g * B, 64), B)]],
                        gbuf.at[pl.ds(b * B, B)], sems_g[b]).wait()

                    @pl.when(g >= 2)
                    def _():
                        pltpu.make_async_copy(
                            obuf.at[pl.ds(b * B, B)],
                            acc.at[dst2_v.at[g - 2]],
                            sems_s[b]).wait()

                    gB = g * B

                    @plsc.parallel_loop(0, B, step=1, unroll=4)
                    def row(i):
                        widx = jnp.broadcast_to(
                            (gB + i).astype(jnp.int32), (16,))
                        ws = plsc.load_gather(w_v, [widx])
                        rr = b * B + i
                        for k in range(8):
                            v = gbuf[rr, pl.ds(k * 16, 16)]
                            obuf[rr, pl.ds(k * 16, 16)] = v * ws

                    pltpu.async_copy(
                        obuf.at[pl.ds(b * B, B)],
                        acc.at[dst2_v.at[g]], sems_s[b], add=True)

                    @pl.when(g + 2 < nb)
                    def _():
                        gather_batch(g + 2, b)
                return 0

            lax.fori_loop(0, nb // 2, batch_pair, 0)

            for b in (0, 1):
                pltpu.make_async_copy(
                    obuf.at[pl.ds(b * B, B)],
                    acc.at[dst2_v.at[nb - 2 + b]], sems_s[b]).wait()

        for ch in range(ch_count):
            zero_stripe()
            for seg in (0, 1):
                load_seg(seg, nbs[seg], den_mode and ch == 0)
                run_seg(nbs[seg], h_refs[ch])
            plsc.subcore_barrier()
            drain_stripe(ch)

        if den_mode:
            pltpu.sync_copy(den_t.at[pl.ds(0, HALF)],
                            den_stage.at[sub, 0, pl.ds(0, HALF)])
            plsc.subcore_barrier()
            cbase = pl.multiple_of(jnp.minimum(sub, 13) * 384, 128)
            for rnd in (0, 1):
                for j in range(8):
                    pltpu.sync_copy(
                        den_stage.at[rnd * 8 + j, 0, pl.ds(cbase, 384)],
                        den_t.at[pl.ds(j * 384, 384)])

                @plsc.parallel_loop(0, 384 // 16, step=1, unroll=2)
                def dred(q):
                    tot = den_t[pl.ds(q * 16, 16)]
                    for j in range(1, 8):
                        tot = tot + den_t[pl.ds(j * 384 + q * 16, 16)]
                    if rnd == 0:
                        den_t[pl.ds(3072 + q * 16, 16)] = tot
                    else:
                        den_t[pl.ds(3072 + q * 16, 16)] = (
                            tot + den_t[pl.ds(3072 + q * 16, 16)])

            @pl.when(sub < 13)
            def _():
                pltpu.sync_copy(
                    den_t.at[pl.ds(3072, 384)],
                    den_hbm.at[pl.ds(
                        pl.multiple_of(base + cbase, 8), 384)])

            @pl.when(sub == 13)
            def _():
                pltpu.sync_copy(
                    den_t.at[pl.ds(3072, 128)],
                    den_hbm.at[pl.ds(
                        pl.multiple_of(base + 4992, 8), 128)])

    mesh = plsc.VectorSubcoreMesh(core_axis_name="c", subcore_axis_name="s")
    if den_mode:
        out_type = (
            jax.ShapeDtypeStruct((n_out * NP, 128), jnp.float32),
            jax.ShapeDtypeStruct((NP,), jnp.float32),
        )
        extra = [pltpu.VMEM((HALF + 16,), jnp.float32)]
        extra_sh = [pltpu.VMEM_SHARED((16, 1, 5376), jnp.float32)]
    else:
        out_type = jax.ShapeDtypeStruct((n_out * NP, 128), jnp.float32)
        extra = []
        extra_sh = []
    return pl.kernel(
        body,
        out_type=out_type,
        mesh=mesh,
        compiler_params=pltpu.CompilerParams(needs_layout_passes=False),
        scratch_types=[
            pltpu.VMEM((NP,), jnp.float32),
            pltpu.VMEM((HALF,), jnp.float32),
            pltpu.VMEM((SEGCAP,), jnp.float32),
            pltpu.VMEM((SEGCAP,), jnp.int32),
            pltpu.VMEM((SEGB, B), jnp.int32),
            pltpu.VMEM((2, 1, 16), jnp.int32),
            pltpu.VMEM((2 * B, 128), jnp.float32),
            pltpu.VMEM((2 * B, 128), jnp.float32),
        ] + extra + [
            pltpu.VMEM_SHARED((HALF + 16, 128), jnp.float32),
        ] + extra_sh + [
            pltpu.SemaphoreType.DMA,
            pltpu.SemaphoreType.DMA,
            pltpu.SemaphoreType.DMA,
            pltpu.SemaphoreType.DMA,
        ],
    )


@jax.jit
def _run(x, src2, dst2, W1ext, b1r, W2ext3, b2):
    zeros = jnp.zeros((STRIPE, 128), jnp.float32)

    srcR, dstR, cnts = _make_route()(src2, dst2)

    h1 = _matmul_chunks(x, W1ext, 5)
    s1 = h1[4, :, 0]
    d1 = h1[4, :, 1]
    agg1_fn = _make_agg(4, True)
    agg1, den1 = agg1_fn(h1[0], h1[1], h1[2], h1[3], s1, d1, srcR, dstR,
                         cnts, zeros)
    agg1 = agg1.reshape(4, NP, 128)
    den1r = den1.reshape(NP // 256, 1, 256)

    out2 = _matmul2(agg1, den1r, b1r, W2ext3)
    s2 = out2[:, 64]
    d2 = out2[:, 65]
    agg2_fn = _make_agg(1, False)
    agg2 = agg2_fn(out2, s2, d2, srcR, dstR, cnts, zeros)
    agg2 = agg2.reshape(1, NP, 128)

    out = _final(agg2, b2)
    return out[:N_NODES]


def kernel(x, edge_index, W1, a1_src, a1_dst, b1, W2, a2_src, a2_dst, b2):
    ei = edge_index.astype(jnp.int32)
    loop = jnp.arange(N_NODES, dtype=jnp.int32)
    padv = N_NODES + (jnp.arange(EP - EL, dtype=jnp.int32) % (NP - N_NODES))
    src = jnp.concatenate([ei[0], loop, jnp.zeros((EP - EL,), jnp.int32)])
    dst = jnp.concatenate([ei[1], loop, padv])
    src2 = src.reshape(EP // B, B)
    dst2 = dst.reshape(EP // B, B)

    xp = jnp.pad(x, ((0, NP - N_NODES), (0, 0)))

    W1ext = jnp.zeros((256, 640), jnp.float32)
    W1ext = W1ext.at[:, :512].set(W1)
    W1ext = W1ext.at[:, 512].set(W1 @ a1_src)
    W1ext = W1ext.at[:, 513].set(W1 @ a1_dst)

    W2ext = jnp.zeros((512, 128), jnp.float32)
    W2ext = W2ext.at[:, :64].set(W2)
    W2ext = W2ext.at[:, 64].set(W2 @ a2_src)
    W2ext = W2ext.at[:, 65].set(W2 @ a2_dst)
    W2ext3 = W2ext.reshape(4, 128, 128)

    b1r = b1.reshape(4, 1, 128)

    return _run(xp, src2, dst2, W1ext, b1r, W2ext3, b2)
